# baseline (speedup 1.0000x reference)
import jax
import jax.numpy as jnp
from jax import lax
from jax.experimental import pallas as pl
from jax.experimental.pallas import tpu as pltpu
from jax.experimental.pallas import tpu_sc as plsc

N_NODES = 800000
N_GRAPHS = 100000
D = 32
B = 4096
L = 200

NC = 2
NS = 16

QS = 25000
QCAP = 25008
DUMP = QCAP
QROWS = QCAP + 8
TBL = 100008
CH = 256
CW = 128
NFC = 196
BW = 128


def _iota16():
  return lax.iota(jnp.int32, 16)


def _p1_body(featsT, seg, nb, table, sp_feat, sp_cnt, bvm,
             segb0, segb1, colb0, colb1, featb0, featb1, idx20, idx21,
             onesb, fbuf, cbuf, lsem0, lsem1, asem0, asem1, zsem):
  c = lax.axis_index("c")
  w = lax.axis_index("s")
  it = _iota16()
  segbs, colbs, featbs = (segb0, segb1), (colb0, colb1), (featb0, featb1)
  idx2s, lsems, asems = (idx20, idx21), (lsem0, lsem1), (asem0, asem1)

  pltpu.sync_copy(nb, bvm)
  bv = bvm[pl.ds(0, 16)]
  b0 = jnp.where(c == 0, bv[0], bv[2])
  b1 = jnp.where(c == 0, bv[1], bv[3])
  b2 = jnp.where(c == 0, bv[2], bv[4])

  def _fill(r, _):
    onesb[r, pl.ds(0, 16)] = jnp.where(it == 0, 1.0, 0.0)
    return 0
  lax.fori_loop(0, CH, _fill, 0)

  flo = (w * NFC) // NS
  fhi = ((w + 1) * NFC) // NS

  def _pass(p_local, _):
    p = c * 2 + p_local
    qlo = p * QS
    qn = jnp.where(p == 3, QCAP, QS)
    nlo = jnp.where(p_local == 0, b0, b1)
    nhi = jnp.where(p_local == 0, b1, b2)

    def _zfill(r, _):
      fbuf[r, pl.ds(0, 16)] = jnp.zeros((16,), jnp.float32)
      fbuf[r, pl.ds(16, 16)] = jnp.zeros((16,), jnp.float32)
      cbuf[r, pl.ds(0, 16)] = jnp.zeros((16,), jnp.float32)
      return 0
    lax.fori_loop(0, CW, _zfill, 0)

    def _zero(i, _):
      st = jnp.minimum(i * CW, QROWS - CW)
      pltpu.async_copy(fbuf, sp_feat.at[pl.ds(st, CW)], zsem)
      pltpu.async_copy(cbuf, sp_cnt.at[pl.ds(st, CW)], zsem)
      return 0
    lax.fori_loop(flo, fhi, _zero, 0)

    def _zdrain(i, _):
      pltpu.make_async_copy(fbuf, sp_feat.at[pl.ds(0, CW)], zsem).wait()
      pltpu.make_async_copy(cbuf, sp_cnt.at[pl.ds(0, CW)], zsem).wait()
      return 0
    lax.fori_loop(flo, fhi, _zdrain, 0)
    plsc.subcore_barrier()

    lo_w = nlo + (nhi - nlo) * w // NS
    hi_w = nlo + (nhi - nlo) * (w + 1) // NS
    a_lo = (lo_w // 8) * 8
    nch = jnp.maximum(0, (hi_w - a_lo + CH - 1) // CH)

    def _fire_load(ci, par):
      st = jnp.minimum(a_lo + ci * CH, N_NODES - CH)
      pltpu.async_copy(seg.at[pl.ds(st, CH)], segbs[par], lsems[par])
      pltpu.async_copy(featsT.at[:, pl.ds(st, CH)], colbs[par], lsems[par])

    def _wait_load(par):
      pltpu.make_async_copy(seg.at[pl.ds(0, CH)], segbs[par],
                            lsems[par]).wait()
      pltpu.make_async_copy(featsT.at[:, pl.ds(0, CH)], colbs[par],
                            lsems[par]).wait()

    def _drain_adds(par):
      for _ in range(CH // CW):
        pltpu.make_async_copy(featbs[par].at[pl.ds(0, CW)],
                              sp_feat.at[pl.ds(0, CW)],
                              asems[par]).wait()
        pltpu.make_async_copy(onesb.at[pl.ds(0, CW)],
                              sp_cnt.at[pl.ds(0, CW)], asems[par]).wait()

    @pl.when(nch > 0)
    def _():
      _fire_load(0, 0)
    @pl.when(nch > 1)
    def _():
      _fire_load(1, 1)

    def _chunk2(i2, _):
      for par in range(2):
        ci = i2 * 2 + par

        @pl.when(ci < nch)
        def _():
          raw = a_lo + ci * CH
          st = jnp.minimum(raw, N_NODES - CH)
          _wait_load(par)
          def _tr(nb16, _):
            n0 = nb16 * 16
            for d0 in (0, 16):
              for j in range(16):
                rot = n0 + ((it + j) & 15)
                g = plsc.load_gather(colbs[par], [d0 + it, rot])
                plsc.store_scatter(featbs[par], [rot, d0 + it], g)
            return 0
          lax.fori_loop(0, CH // 16, _tr, 0)
          for k in range(CH // 16):
            v = segbs[par][pl.ds(k * 16, 16)]
            pos = st + k * 16 + it
            valid = (pos >= lo_w) & (pos >= raw) & (pos < hi_w)
            idx = jnp.where(valid, v - qlo, DUMP)
            idx2s[par][k // 8, pl.ds((k % 8) * 16, 16)] = idx
          for j in range(CH // CW):
            pltpu.async_copy(featbs[par].at[pl.ds(j * CW, CW)],
                             sp_feat.at[idx2s[par].at[j]],
                             asems[par], add=True)
            pltpu.async_copy(onesb.at[pl.ds(j * CW, CW)],
                             sp_cnt.at[idx2s[par].at[j]],
                             asems[par], add=True)

        @pl.when(ci + 2 < nch)
        def _():
          _drain_adds(par)
          _fire_load(ci + 2, par)
      return 0
    lax.fori_loop(0, (nch + 1) // 2, _chunk2, 0)

    for par in range(2):
      @pl.when(nch >= (2 if par else 1))
      def _(par=par):
        _drain_adds(par)
    plsc.subcore_barrier()

    def _fin(i, _):
      st = jnp.minimum(i * CW, qn - CW)
      pltpu.sync_copy(sp_feat.at[pl.ds(st, CW)], fbuf)
      pltpu.sync_copy(sp_cnt.at[pl.ds(st, CW)], cbuf)
      for g in range(CW):
        crow = cbuf[g, pl.ds(0, 16)]
        inv = 1.0 / jnp.maximum(crow, 1.0)
        sp = jnp.full((16,), inv[0])
        fbuf[g, pl.ds(0, 16)] = fbuf[g, pl.ds(0, 16)] * sp
        fbuf[g, pl.ds(16, 16)] = fbuf[g, pl.ds(16, 16)] * sp
      pltpu.sync_copy(fbuf, table.at[pl.ds(qlo + st, CW)])
      return 0
    lax.fori_loop(flo, fhi, _fin, 0)
    plsc.subcore_barrier()
    return 0

  lax.fori_loop(0, 2, _pass, 0)


def _p2_body(table, seqT, out5, sidx0, sidx1, idx10, idx11, rows0, rows1,
             tr0, tr1, lsem0, lsem1, gsem0, gsem1, osem0, osem1):
  c = lax.axis_index("c")
  w = lax.axis_index("s")
  wid = w * NC + c
  it = _iota16()
  sidxs, idx1s, rowss = (sidx0, sidx1), (idx10, idx11), (rows0, rows1)
  trs = (tr0, tr1)
  lsems, gsems, osems = (lsem0, lsem1), (gsem0, gsem1), (osem0, osem1)

  def _fire_load(li, par):
    pltpu.async_copy(seqT.at[pl.ds(li * B + wid * BW, BW)], sidxs[par],
                     lsems[par])

  _fire_load(0, 0)
  _fire_load(1, 1)

  def _stage_a(li, par):
    pltpu.make_async_copy(seqT.at[pl.ds(0, BW)], sidxs[par],
                          lsems[par]).wait()
    for k in range(BW // 16):
      v = sidxs[par][pl.ds(k * 16, 16)] - 1
      v = jnp.where(v < 0, N_GRAPHS, v)
      idx1s[par][pl.ds(k * 16, 16)] = v

    @pl.when(li + 2 < L)
    def _():
      _fire_load(li + 2, par)
    pltpu.async_copy(table.at[idx1s[par]], rowss[par], gsems[par])

  def _stage_b(li, par):
    pltpu.make_async_copy(table.at[pl.ds(0, BW)], rowss[par],
                          gsems[par]).wait()

    @pl.when(li >= 2)
    def _():
      for r in range(4):
        pltpu.make_async_copy(trs[par].at[r], out5.at[0, 0, 0],
                              osems[par]).wait()
    def _tr(b16, _):
      b0 = b16 * 16
      for d0 in (0, 16):
        dv = d0 + it
        for j in range(16):
          rot = b0 + ((it + j) & 15)
          g = plsc.load_gather(rowss[par], [rot, dv])
          plsc.store_scatter(trs[par],
                             [dv >> 3, dv & 7, rot], g)
      return 0
    lax.fori_loop(0, BW // 16, _tr, 0)
    for r in range(4):
      pltpu.async_copy(trs[par].at[r], out5.at[li, r, wid], osems[par])

  def _chunk2(i2, _):
    for par in range(2):
      li = i2 * 2 + par
      _stage_a(li, par)
      lj = li - 1

      @pl.when(lj >= 0)
      def _():
        _stage_b(lj, 1 - par)
    return 0
  lax.fori_loop(0, L // 2, _chunk2, 0)
  _stage_b(L - 1, (L - 1) & 1)
  for par in range(2):
    for r in range(4):
      pltpu.make_async_copy(trs[par].at[r], out5.at[0, 0, 0],
                            osems[par]).wait()


def kernel(node_feats, segment_ids, sequence):
  seg = segment_ids.astype(jnp.int32)
  seqT = sequence.astype(jnp.int32).T.reshape(-1)
  featsT = node_feats.T
  nb1 = jnp.sum((seg < QS).astype(jnp.int32))
  nb2 = jnp.sum((seg < 2 * QS).astype(jnp.int32))
  nb3 = jnp.sum((seg < 3 * QS).astype(jnp.int32))
  nbv = jnp.zeros((16,), jnp.int32)
  nbv = nbv.at[1].set(nb1).at[2].set(nb2).at[3].set(nb3).at[4].set(N_NODES)

  mesh = plsc.VectorSubcoreMesh(core_axis_name="c", subcore_axis_name="s",
                                num_cores=NC, num_subcores=NS)
  cparams = pltpu.CompilerParams(use_tc_tiling_on_sc=False,
                                 needs_layout_passes=False)

  p1 = pl.kernel(
      _p1_body,
      out_type=jax.ShapeDtypeStruct((TBL, D), jnp.float32),
      mesh=mesh,
      compiler_params=cparams,
      scratch_types=[
          pltpu.VMEM_SHARED((QROWS, D), jnp.float32),
          pltpu.VMEM_SHARED((QROWS, 16), jnp.float32),
          pltpu.VMEM((16,), jnp.int32),
          pltpu.VMEM((CH,), jnp.int32),
          pltpu.VMEM((CH,), jnp.int32),
          pltpu.VMEM((D, CH), jnp.float32),
          pltpu.VMEM((D, CH), jnp.float32),
          pltpu.VMEM((CH, D), jnp.float32),
          pltpu.VMEM((CH, D), jnp.float32),
          pltpu.VMEM((CH // CW, CW), jnp.int32),
          pltpu.VMEM((CH // CW, CW), jnp.int32),
          pltpu.VMEM((CH, 16), jnp.float32),
          pltpu.VMEM((CW, D), jnp.float32),
          pltpu.VMEM((CW, 16), jnp.float32),
          pltpu.SemaphoreType.DMA,
          pltpu.SemaphoreType.DMA,
          pltpu.SemaphoreType.DMA,
          pltpu.SemaphoreType.DMA,
          pltpu.SemaphoreType.DMA,
      ],
  )
  table = p1(featsT, seg, nbv)

  p2 = pl.kernel(
      _p2_body,
      out_type=jax.ShapeDtypeStruct((L, 4, B // BW, 8, BW), jnp.float32),
      mesh=mesh,
      compiler_params=cparams,
      scratch_types=[
          pltpu.VMEM((BW,), jnp.int32),
          pltpu.VMEM((BW,), jnp.int32),
          pltpu.VMEM((BW,), jnp.int32),
          pltpu.VMEM((BW,), jnp.int32),
          pltpu.VMEM((BW, D), jnp.float32),
          pltpu.VMEM((BW, D), jnp.float32),
          pltpu.VMEM((4, 8, BW), jnp.float32),
          pltpu.VMEM((4, 8, BW), jnp.float32),
          pltpu.SemaphoreType.DMA,
          pltpu.SemaphoreType.DMA,
          pltpu.SemaphoreType.DMA,
          pltpu.SemaphoreType.DMA,
          pltpu.SemaphoreType.DMA,
          pltpu.SemaphoreType.DMA,
      ],
  )
  out5 = p2(table, seqT)
  return out5.transpose(2, 4, 0, 1, 3).reshape(B, L, D)
